# Initial kernel scaffold; baseline (speedup 1.0000x reference)
import jax
import jax.numpy as jnp
from jax.experimental import pallas as pl


def kernel(values_add, values_query):
    raise NotImplementedError("write your pallas kernel here")


# speedup vs baseline: 5.7863x; 5.7863x over previous
import functools

import jax
import jax.numpy as jnp
from jax import lax
from jax.experimental import pallas as pl
from jax.experimental.pallas import tpu as pltpu
from jax.experimental.pallas import tpu_sc as plsc

NBITS = 1 << 24
MASK = NBITS - 1
PRIME_I32 = 2654435761 - (1 << 32)

N_ADD = 1_000_000
N_ADD_PAD = 1 << 20
N_Q = 1 << 22

NUM_CORES = 2
NUM_SUBCORES = 16
NW = NUM_CORES * NUM_SUBCORES
LB = 128
CR = 8

def _hash_rows(vals_ref, idx_ref):
  for j in range(CR):
    for l in range(LB // 16):
      v = vals_ref[j, pl.ds(l * 16, 16)]
      idx_ref[j, pl.ds(l * 16, 16)] = (v * jnp.int32(PRIME_I32)) & jnp.int32(MASK)


def _scatter_body(vals_hbm, s_hbm, vals_v, idx_v, ones_v, sem):
  wid = lax.axis_index("s") * NUM_CORES + lax.axis_index("c")
  rows_per_tile = N_ADD_PAD // LB // NW
  row0 = wid * rows_per_tile
  for l in range(LB // 16):
    ones_v[pl.ds(l * 16, 16)] = jnp.full((16,), 1, jnp.int32)

  @pl.loop(0, rows_per_tile // CR)
  def _chunk(ci):
    r = row0 + ci * CR
    pltpu.sync_copy(vals_hbm.at[pl.ds(r, CR)], vals_v)
    _hash_rows(vals_v, idx_v)
    copies = [
        pltpu.async_copy(ones_v, s_hbm.at[idx_v.at[j]], sem) for j in range(CR)
    ]
    for cp in copies:
      cp.wait()


def _gather_body(qvals_hbm, w_hbm, out_hbm, qv, qidx, res, sem):
  wid = lax.axis_index("s") * NUM_CORES + lax.axis_index("c")
  rows_per_tile = N_Q // LB // NW
  row0 = wid * rows_per_tile

  @pl.loop(0, rows_per_tile // CR)
  def _chunk(ci):
    r = row0 + ci * CR
    pltpu.sync_copy(qvals_hbm.at[pl.ds(r, CR)], qv)
    _hash_rows(qv, qidx)
    copies = [
        pltpu.async_copy(w_hbm.at[qidx.at[j]], res.at[j], sem)
        for j in range(CR)
    ]
    for cp in copies:
      cp.wait()
    pltpu.sync_copy(res, out_hbm.at[pl.ds(r, CR)])


@functools.cache
def _sc_kernels():
  mesh = plsc.VectorSubcoreMesh(
      core_axis_name="c", subcore_axis_name="s",
      num_cores=NUM_CORES, num_subcores=NUM_SUBCORES)
  scatter = pl.kernel(
      _scatter_body,
      mesh=mesh,
      scratch_types=[
          pltpu.VMEM((CR, LB), jnp.int32),
          pltpu.VMEM((CR, LB), jnp.int32),
          pltpu.VMEM((LB,), jnp.int32),
          pltpu.SemaphoreType.DMA,
      ],
  )
  gather = pl.kernel(
      _gather_body,
      out_type=jax.ShapeDtypeStruct((N_Q // LB, LB), jnp.int32),
      mesh=mesh,
      scratch_types=[
          pltpu.VMEM((CR, LB), jnp.int32),
          pltpu.VMEM((CR, LB), jnp.int32),
          pltpu.VMEM((CR, LB), jnp.int32),
          pltpu.SemaphoreType.DMA,
      ],
  )
  return scatter, gather


R2D = 16384
C2D = 1024
RB = 512
NBLK = R2D // RB


def _window_body(x_ref, prev_ref, next_ref, o_ref):
  X = x_ref[...]
  Xe = jnp.concatenate([prev_ref[0], X, next_ref[0]], axis=0)
  tail = jnp.concatenate([Xe[:1, C2D - 8:], Xe[:-1, C2D - 8:]], axis=0)
  E1 = jnp.concatenate([tail, Xe], axis=1)
  A = E1[:, 8:]
  for e in range(1, 7):
    A = A | E1[:, 8 - e:8 - e + C2D]
  head = jnp.concatenate([A[1:, :8], A[-1:, :8]], axis=0)
  E2 = jnp.concatenate([A, head], axis=1)
  W = E2[:, :C2D]
  for d in range(1, 7):
    W = W & E2[:, d:d + C2D]
  o_ref[...] = W[1:RB + 1]


_window = pl.pallas_call(
    _window_body,
    grid=(NBLK,),
    in_specs=[
        pl.BlockSpec((RB, C2D), lambda i: (i, 0)),
        pl.BlockSpec((1, 1, C2D), lambda i: (i, 0, 0)),
        pl.BlockSpec((1, 1, C2D), lambda i: (i, 0, 0)),
    ],
    out_specs=pl.BlockSpec((RB, C2D), lambda i: (i, 0)),
    out_shape=jax.ShapeDtypeStruct((R2D, C2D), jnp.int32),
)


def kernel(values_add, values_query):
  pad = jnp.broadcast_to(values_add[0], (N_ADD_PAD - N_ADD,))
  vals2d = jnp.concatenate([values_add, pad]).reshape(N_ADD_PAD // LB, LB)
  q2d = values_query.reshape(N_Q // LB, LB)

  scatter_markers, gather_queries = _sc_kernels()
  s_ref = jax.new_ref(jnp.zeros((NBITS,), jnp.int32))
  scatter_markers(vals2d, s_ref)
  S2 = s_ref[...].reshape(R2D, C2D)

  prev_rows = jnp.roll(S2[RB - 1::RB], 1, axis=0).reshape(NBLK, 1, C2D)
  next_rows = jnp.roll(S2[0::RB], -1, axis=0).reshape(NBLK, 1, C2D)
  W2 = _window(S2, prev_rows, next_rows)

  out2d = gather_queries(q2d, W2.reshape(-1))
  return out2d.reshape(-1) != 0
